# initial kernel scaffold (unmeasured)
import jax
import jax.numpy as jnp
from jax import lax
from jax.experimental import pallas as pl
from jax.experimental.pallas import tpu as pltpu

N_DEV = 4


def kernel(x, W1, W2):
    m, k = x.shape
    n = W2.shape[1]
    assert m % N_DEV == 0
    chunk = m // N_DEV

    def body(x_ref, w1_ref, w2_ref, out_ref, rs_tmp,
             rs_send, rs_recv, ag_send, ag_recv):
        my_pos = lax.axis_index("i")
        left = (my_pos + N_DEV - 1) % N_DEV
        right = (my_pos + 1) % N_DEV

        h = jnp.maximum(
            jnp.dot(x_ref[...], w1_ref[...],
                    preferred_element_type=jnp.float32),
            0.0,
        )
        out_ref[...] = jnp.dot(h, w2_ref[...],
                               preferred_element_type=jnp.float32)

        barrier_sem = pltpu.get_barrier_semaphore()
        for nbr in (left, right):
            pl.semaphore_signal(
                barrier_sem, inc=1,
                device_id=(nbr,), device_id_type=pl.DeviceIdType.MESH,
            )
        pl.semaphore_wait(barrier_sem, 2)

        for h in range(N_DEV - 1):
            send_c = (my_pos + N_DEV - h) % N_DEV
            recv_c = (my_pos + N_DEV - 1 - h) % N_DEV
            rdma = pltpu.make_async_remote_copy(
                src_ref=out_ref.at[pl.ds(send_c * chunk, chunk), :],
                dst_ref=rs_tmp.at[h],
                send_sem=rs_send.at[h],
                recv_sem=rs_recv.at[h],
                device_id=(right,),
                device_id_type=pl.DeviceIdType.MESH,
            )
            rdma.start()
            rdma.wait()
            sl = pl.ds(recv_c * chunk, chunk)
            out_ref[sl, :] = out_ref[sl, :] + rs_tmp[h]

        for g in range(N_DEV - 1):
            send_c = (my_pos + 1 + N_DEV - g) % N_DEV
            rdma = pltpu.make_async_remote_copy(
                src_ref=out_ref.at[pl.ds(send_c * chunk, chunk), :],
                dst_ref=out_ref.at[pl.ds(send_c * chunk, chunk), :],
                send_sem=ag_send.at[g],
                recv_sem=ag_recv.at[g],
                device_id=(right,),
                device_id_type=pl.DeviceIdType.MESH,
            )
            rdma.start()
            rdma.wait()

    return pl.pallas_call(
        body,
        out_shape=jax.ShapeDtypeStruct((m, n), jnp.float32),
        in_specs=[
            pl.BlockSpec(memory_space=pltpu.VMEM),
            pl.BlockSpec(memory_space=pltpu.VMEM),
            pl.BlockSpec(memory_space=pltpu.VMEM),
        ],
        out_specs=pl.BlockSpec(memory_space=pltpu.VMEM),
        scratch_shapes=[
            pltpu.VMEM((N_DEV - 1, chunk, n), jnp.float32),
            pltpu.SemaphoreType.DMA((N_DEV - 1,)),
            pltpu.SemaphoreType.DMA((N_DEV - 1,)),
            pltpu.SemaphoreType.DMA((N_DEV - 1,)),
            pltpu.SemaphoreType.DMA((N_DEV - 1,)),
        ],
        compiler_params=pltpu.CompilerParams(collective_id=0),
    )(x, W1, W2)


# baseline (device time: 232248 ns/iter reference)
import jax
import jax.numpy as jnp
from jax import lax
from jax.experimental import pallas as pl
from jax.experimental.pallas import tpu as pltpu

N_DEV = 4
H_TILES = 4


def _mlp_partial(x, W1, W2):
    m, k = x.shape
    h_per, n = W2.shape
    ht = h_per // H_TILES

    def body(x_ref, w1_ref, w2_ref, out_ref):
        t = pl.program_id(0)
        h = jnp.maximum(
            jnp.dot(x_ref[...], w1_ref[...],
                    preferred_element_type=jnp.float32),
            0.0,
        )
        p = jnp.dot(h, w2_ref[...], preferred_element_type=jnp.float32)

        @pl.when(t == 0)
        def _():
            out_ref[...] = p

        @pl.when(t != 0)
        def _():
            out_ref[...] = out_ref[...] + p

    return pl.pallas_call(
        body,
        grid=(H_TILES,),
        out_shape=jax.ShapeDtypeStruct((m, n), jnp.float32),
        in_specs=[
            pl.BlockSpec((m, k), lambda t: (0, 0)),
            pl.BlockSpec((k, ht), lambda t: (0, t)),
            pl.BlockSpec((ht, n), lambda t: (t, 0)),
        ],
        out_specs=pl.BlockSpec((m, n), lambda t: (0, 0)),
        compiler_params=pltpu.CompilerParams(
            vmem_limit_bytes=100 * 1024 * 1024,
        ),
    )(x, W1, W2)


def _ring_allreduce(partial):
    m, n = partial.shape
    chunk = m // N_DEV

    def body(p_ref, out_ref, rs_tmp, rs_send, rs_recv, ag_send, ag_recv):
        my_pos = lax.axis_index("i")
        left = (my_pos + N_DEV - 1) % N_DEV
        right = (my_pos + 1) % N_DEV

        out_ref[...] = p_ref[...]

        barrier_sem = pltpu.get_barrier_semaphore()
        for nbr in (left, right):
            pl.semaphore_signal(
                barrier_sem, inc=1,
                device_id=(nbr,), device_id_type=pl.DeviceIdType.MESH,
            )
        pl.semaphore_wait(barrier_sem, 2)

        for h in range(N_DEV - 1):
            send_c = (my_pos + N_DEV - h) % N_DEV
            recv_c = (my_pos + N_DEV - 1 - h) % N_DEV
            rdma = pltpu.make_async_remote_copy(
                src_ref=out_ref.at[pl.ds(send_c * chunk, chunk), :],
                dst_ref=rs_tmp.at[h],
                send_sem=rs_send.at[h],
                recv_sem=rs_recv.at[h],
                device_id=(right,),
                device_id_type=pl.DeviceIdType.MESH,
            )
            rdma.start()
            rdma.wait()
            sl = pl.ds(recv_c * chunk, chunk)
            out_ref[sl, :] = out_ref[sl, :] + rs_tmp[h]

        for g in range(N_DEV - 1):
            send_c = (my_pos + 1 + N_DEV - g) % N_DEV
            rdma = pltpu.make_async_remote_copy(
                src_ref=out_ref.at[pl.ds(send_c * chunk, chunk), :],
                dst_ref=out_ref.at[pl.ds(send_c * chunk, chunk), :],
                send_sem=ag_send.at[g],
                recv_sem=ag_recv.at[g],
                device_id=(right,),
                device_id_type=pl.DeviceIdType.MESH,
            )
            rdma.start()
            rdma.wait()

    return pl.pallas_call(
        body,
        out_shape=jax.ShapeDtypeStruct((m, n), jnp.float32),
        in_specs=[pl.BlockSpec(memory_space=pltpu.VMEM)],
        out_specs=pl.BlockSpec(memory_space=pltpu.VMEM),
        scratch_shapes=[
            pltpu.VMEM((N_DEV - 1, chunk, n), jnp.float32),
            pltpu.SemaphoreType.DMA((N_DEV - 1,)),
            pltpu.SemaphoreType.DMA((N_DEV - 1,)),
            pltpu.SemaphoreType.DMA((N_DEV - 1,)),
            pltpu.SemaphoreType.DMA((N_DEV - 1,)),
        ],
        compiler_params=pltpu.CompilerParams(
            collective_id=0,
            vmem_limit_bytes=100 * 1024 * 1024,
        ),
    )(partial)


def kernel(x, W1, W2):
    partial = _mlp_partial(x, W1, W2)
    return _ring_allreduce(partial)


# device time: 196336 ns/iter; 1.1829x vs baseline; 1.1829x over previous
import jax
import jax.numpy as jnp
from jax import lax
from jax.experimental import pallas as pl
from jax.experimental.pallas import tpu as pltpu

N_DEV = 4
R = 4
T = 4


def kernel(x, W1, W2):
    m, k = x.shape
    h_per, n = W2.shape
    chunk = m // N_DEV
    ht = h_per // T

    def body(x_ref, w1_ref, w2_ref, out_ref, rs_tmp,
             rs_send, rs_recv, ag_send, ag_recv):
        my_pos = lax.axis_index("i")
        left = (my_pos + N_DEV - 1) % N_DEV
        right = (my_pos + 1) % N_DEV
        r = pl.program_id(0)
        t = pl.program_id(1)

        chunk_id = (my_pos + N_DEV - r) % N_DEV
        rows = pl.ds(chunk_id * chunk, chunk)
        h = jnp.maximum(
            jnp.dot(x_ref[rows, :], w1_ref[...],
                    preferred_element_type=jnp.float32),
            0.0,
        )
        p = jnp.dot(h, w2_ref[...], preferred_element_type=jnp.float32)

        @pl.when(t == 0)
        def _():
            out_ref[rows, :] = p

        @pl.when(t != 0)
        def _():
            out_ref[rows, :] = out_ref[rows, :] + p

        def rs_hop(h_idx, send_c):
            rdma = pltpu.make_async_remote_copy(
                src_ref=out_ref.at[pl.ds(send_c * chunk, chunk), :],
                dst_ref=rs_tmp.at[h_idx],
                send_sem=rs_send.at[h_idx],
                recv_sem=rs_recv.at[h_idx],
                device_id=(right,),
                device_id_type=pl.DeviceIdType.MESH,
            )
            return rdma

        def rs_wait_and_add(h_idx, recv_c):
            rs_hop(h_idx, recv_c).wait()
            sl = pl.ds(recv_c * chunk, chunk)
            out_ref[sl, :] = out_ref[sl, :] + rs_tmp[h_idx]

        @pl.when(jnp.logical_and(r == 0, t == T - 1))
        def _():
            barrier_sem = pltpu.get_barrier_semaphore()
            for nbr in (left, right):
                pl.semaphore_signal(
                    barrier_sem, inc=1,
                    device_id=(nbr,), device_id_type=pl.DeviceIdType.MESH,
                )
            pl.semaphore_wait(barrier_sem, 2)
            rs_hop(0, my_pos).start()

        @pl.when(jnp.logical_and(r == 1, t == T - 1))
        def _():
            c = (my_pos + N_DEV - 1) % N_DEV
            rs_wait_and_add(0, c)
            rs_hop(1, c).start()

        @pl.when(jnp.logical_and(r == 2, t == T - 1))
        def _():
            c = (my_pos + N_DEV - 2) % N_DEV
            rs_wait_and_add(1, c)
            rs_hop(2, c).start()

        @pl.when(jnp.logical_and(r == 3, t == T - 1))
        def _():
            rs_wait_and_add(2, (my_pos + 1) % N_DEV)

            for g in range(N_DEV - 1):
                send_c = (my_pos + 1 + N_DEV - g) % N_DEV
                rdma = pltpu.make_async_remote_copy(
                    src_ref=out_ref.at[pl.ds(send_c * chunk, chunk), :],
                    dst_ref=out_ref.at[pl.ds(send_c * chunk, chunk), :],
                    send_sem=ag_send.at[g],
                    recv_sem=ag_recv.at[g],
                    device_id=(right,),
                    device_id_type=pl.DeviceIdType.MESH,
                )
                rdma.start()
                rdma.wait()

    return pl.pallas_call(
        body,
        grid=(R, T),
        out_shape=jax.ShapeDtypeStruct((m, n), jnp.float32),
        in_specs=[
            pl.BlockSpec((m, k), lambda r, t: (0, 0)),
            pl.BlockSpec((k, ht), lambda r, t: (0, t)),
            pl.BlockSpec((ht, n), lambda r, t: (t, 0)),
        ],
        out_specs=pl.BlockSpec((m, n), lambda r, t: (0, 0)),
        scratch_shapes=[
            pltpu.VMEM((N_DEV - 1, chunk, n), jnp.float32),
            pltpu.SemaphoreType.DMA((N_DEV - 1,)),
            pltpu.SemaphoreType.DMA((N_DEV - 1,)),
            pltpu.SemaphoreType.DMA((N_DEV - 1,)),
            pltpu.SemaphoreType.DMA((N_DEV - 1,)),
        ],
        compiler_params=pltpu.CompilerParams(
            collective_id=0,
            vmem_limit_bytes=100 * 1024 * 1024,
            dimension_semantics=("arbitrary", "arbitrary"),
        ),
    )(x, W1, W2)


# device time: 134259 ns/iter; 1.7299x vs baseline; 1.4624x over previous
import jax
import jax.numpy as jnp
from jax import lax
from jax.experimental import pallas as pl
from jax.experimental.pallas import tpu as pltpu

N_DEV = 4
R = 4
T = 4


def kernel(x, W1, W2):
    m, k = x.shape
    h_per, n = W2.shape
    chunk = m // N_DEV
    half = chunk // 2
    ht = h_per // T

    def body(x_ref, w1_ref, w2_ref, out_ref,
             cw_tmp, ccw_tmp,
             cw_rs_s, cw_rs_r, ccw_rs_s, ccw_rs_r,
             cw_ag_s, cw_ag_r, ccw_ag_s, ccw_ag_r):
        my_pos = lax.axis_index("i")
        left = (my_pos + N_DEV - 1) % N_DEV
        right = (my_pos + 1) % N_DEV
        r = pl.program_id(0)
        t = pl.program_id(1)

        def top(c):
            return pl.ds(c * chunk, half)

        def bot(c):
            return pl.ds(c * chunk + half, half)

        off = jnp.where(r == 1, 3, jnp.where(r == 2, 1, jnp.where(r == 3, 2, 0)))
        chunk_id = (my_pos + off) % N_DEV
        rows = pl.ds(chunk_id * chunk, chunk)
        h = jnp.maximum(
            jnp.dot(x_ref[rows, :], w1_ref[...],
                    preferred_element_type=jnp.float32),
            0.0,
        )
        p = jnp.dot(h, w2_ref[...], preferred_element_type=jnp.float32)

        @pl.when(t == 0)
        def _():
            out_ref[rows, :] = p

        @pl.when(t != 0)
        def _():
            out_ref[rows, :] = out_ref[rows, :] + p

        def cw_hop(i, c):
            return pltpu.make_async_remote_copy(
                src_ref=out_ref.at[top(c), :],
                dst_ref=cw_tmp.at[i],
                send_sem=cw_rs_s.at[i],
                recv_sem=cw_rs_r.at[i],
                device_id=(right,),
                device_id_type=pl.DeviceIdType.MESH,
            )

        def ccw_hop(i, c):
            return pltpu.make_async_remote_copy(
                src_ref=out_ref.at[bot(c), :],
                dst_ref=ccw_tmp.at[i],
                send_sem=ccw_rs_s.at[i],
                recv_sem=ccw_rs_r.at[i],
                device_id=(left,),
                device_id_type=pl.DeviceIdType.MESH,
            )

        def cw_add(i, c):
            cw_hop(i, c).wait()
            out_ref[top(c), :] = out_ref[top(c), :] + cw_tmp[i]

        def ccw_add(i, c):
            ccw_hop(i, c).wait()
            out_ref[bot(c), :] = out_ref[bot(c), :] + ccw_tmp[i]

        @pl.when(jnp.logical_and(r == 0, t == T - 1))
        def _():
            barrier_sem = pltpu.get_barrier_semaphore()
            for nbr in (left, right):
                pl.semaphore_signal(
                    barrier_sem, inc=1,
                    device_id=(nbr,), device_id_type=pl.DeviceIdType.MESH,
                )
            pl.semaphore_wait(barrier_sem, 2)
            cw_hop(0, my_pos).start()
            ccw_hop(0, my_pos).start()

        @pl.when(jnp.logical_and(r == 1, t == T - 1))
        def _():
            c = (my_pos + N_DEV - 1) % N_DEV
            cw_add(0, c)
            cw_hop(1, c).start()

        @pl.when(jnp.logical_and(r == 2, t == T - 1))
        def _():
            c = (my_pos + 1) % N_DEV
            ccw_add(0, c)
            ccw_hop(1, c).start()

        @pl.when(jnp.logical_and(r == 3, t == T - 1))
        def _():
            c2 = (my_pos + 2) % N_DEV
            cw_add(1, c2)
            cw_hop(2, c2).start()
            ccw_add(1, c2)
            ccw_hop(2, c2).start()

            c_top = (my_pos + 1) % N_DEV
            c_bot = (my_pos + N_DEV - 1) % N_DEV
            cw_add(2, c_top)
            ccw_add(2, c_bot)

            for g in range(N_DEV - 1):
                cw_c = (my_pos + 1 + N_DEV - g) % N_DEV
                ccw_c = (my_pos + N_DEV - 1 + g) % N_DEV
                cw = pltpu.make_async_remote_copy(
                    src_ref=out_ref.at[top(cw_c), :],
                    dst_ref=out_ref.at[top(cw_c), :],
                    send_sem=cw_ag_s.at[g],
                    recv_sem=cw_ag_r.at[g],
                    device_id=(right,),
                    device_id_type=pl.DeviceIdType.MESH,
                )
                ccw = pltpu.make_async_remote_copy(
                    src_ref=out_ref.at[bot(ccw_c), :],
                    dst_ref=out_ref.at[bot(ccw_c), :],
                    send_sem=ccw_ag_s.at[g],
                    recv_sem=ccw_ag_r.at[g],
                    device_id=(left,),
                    device_id_type=pl.DeviceIdType.MESH,
                )
                cw.start()
                ccw.start()
                cw.wait()
                ccw.wait()

    return pl.pallas_call(
        body,
        grid=(R, T),
        out_shape=jax.ShapeDtypeStruct((m, n), jnp.float32),
        in_specs=[
            pl.BlockSpec((m, k), lambda r, t: (0, 0)),
            pl.BlockSpec((k, ht), lambda r, t: (0, t)),
            pl.BlockSpec((ht, n), lambda r, t: (t, 0)),
        ],
        out_specs=pl.BlockSpec((m, n), lambda r, t: (0, 0)),
        scratch_shapes=[
            pltpu.VMEM((N_DEV - 1, half, n), jnp.float32),
            pltpu.VMEM((N_DEV - 1, half, n), jnp.float32),
            pltpu.SemaphoreType.DMA((N_DEV - 1,)),
            pltpu.SemaphoreType.DMA((N_DEV - 1,)),
            pltpu.SemaphoreType.DMA((N_DEV - 1,)),
            pltpu.SemaphoreType.DMA((N_DEV - 1,)),
            pltpu.SemaphoreType.DMA((N_DEV - 1,)),
            pltpu.SemaphoreType.DMA((N_DEV - 1,)),
            pltpu.SemaphoreType.DMA((N_DEV - 1,)),
            pltpu.SemaphoreType.DMA((N_DEV - 1,)),
        ],
        compiler_params=pltpu.CompilerParams(
            collective_id=0,
            vmem_limit_bytes=100 * 1024 * 1024,
            dimension_semantics=("arbitrary", "arbitrary"),
        ),
    )(x, W1, W2)


# device time: 133845 ns/iter; 1.7352x vs baseline; 1.0031x over previous
import jax
import jax.numpy as jnp
from jax import lax
from jax.experimental import pallas as pl
from jax.experimental.pallas import tpu as pltpu

N_DEV = 4
R = 4
T = 4


def kernel(x, W1, W2):
    m, k = x.shape
    h_per, n = W2.shape
    chunk = m // N_DEV
    half = chunk // 2
    ht = h_per // T

    def body(x_ref, w1_ref, w2_ref, out_ref,
             cw_tmp, ccw_tmp,
             cw_rs_s, cw_rs_r, ccw_rs_s, ccw_rs_r,
             cw_ag_s, cw_ag_r, ccw_ag_s, ccw_ag_r):
        my_pos = lax.axis_index("i")
        left = (my_pos + N_DEV - 1) % N_DEV
        right = (my_pos + 1) % N_DEV
        r = pl.program_id(0)
        t = pl.program_id(1)

        def top(c):
            return pl.ds(c * chunk, half)

        def bot(c):
            return pl.ds(c * chunk + half, half)

        off = jnp.where(r == 1, 3, jnp.where(r == 2, 1, jnp.where(r == 3, 2, 0)))
        chunk_id = (my_pos + off) % N_DEV
        rows = pl.ds(chunk_id * chunk, chunk)
        h = jnp.maximum(
            jnp.dot(x_ref[rows, :].astype(jnp.bfloat16),
                    w1_ref[...].astype(jnp.bfloat16),
                    preferred_element_type=jnp.float32),
            0.0,
        )
        p = jnp.dot(h.astype(jnp.bfloat16),
                    w2_ref[...].astype(jnp.bfloat16),
                    preferred_element_type=jnp.float32)

        @pl.when(t == 0)
        def _():
            out_ref[rows, :] = p

        @pl.when(t != 0)
        def _():
            out_ref[rows, :] = out_ref[rows, :] + p

        def cw_hop(i, c):
            return pltpu.make_async_remote_copy(
                src_ref=out_ref.at[top(c), :],
                dst_ref=cw_tmp.at[i],
                send_sem=cw_rs_s.at[i],
                recv_sem=cw_rs_r.at[i],
                device_id=(right,),
                device_id_type=pl.DeviceIdType.MESH,
            )

        def ccw_hop(i, c):
            return pltpu.make_async_remote_copy(
                src_ref=out_ref.at[bot(c), :],
                dst_ref=ccw_tmp.at[i],
                send_sem=ccw_rs_s.at[i],
                recv_sem=ccw_rs_r.at[i],
                device_id=(left,),
                device_id_type=pl.DeviceIdType.MESH,
            )

        def cw_add(i, c):
            cw_hop(i, c).wait()
            out_ref[top(c), :] = out_ref[top(c), :] + cw_tmp[i]

        def ccw_add(i, c):
            ccw_hop(i, c).wait()
            out_ref[bot(c), :] = out_ref[bot(c), :] + ccw_tmp[i]

        @pl.when(jnp.logical_and(r == 0, t == T - 1))
        def _():
            barrier_sem = pltpu.get_barrier_semaphore()
            for nbr in (left, right):
                pl.semaphore_signal(
                    barrier_sem, inc=1,
                    device_id=(nbr,), device_id_type=pl.DeviceIdType.MESH,
                )
            pl.semaphore_wait(barrier_sem, 2)
            cw_hop(0, my_pos).start()
            ccw_hop(0, my_pos).start()

        @pl.when(jnp.logical_and(r == 1, t == T - 1))
        def _():
            c = (my_pos + N_DEV - 1) % N_DEV
            cw_add(0, c)
            cw_hop(1, c).start()

        @pl.when(jnp.logical_and(r == 2, t == T - 1))
        def _():
            c = (my_pos + 1) % N_DEV
            ccw_add(0, c)
            ccw_hop(1, c).start()

        @pl.when(jnp.logical_and(r == 3, t == T - 1))
        def _():
            c2 = (my_pos + 2) % N_DEV
            cw_add(1, c2)
            cw_hop(2, c2).start()
            ccw_add(1, c2)
            ccw_hop(2, c2).start()

            c_top = (my_pos + 1) % N_DEV
            c_bot = (my_pos + N_DEV - 1) % N_DEV
            cw_add(2, c_top)
            ccw_add(2, c_bot)

            for g in range(N_DEV - 1):
                cw_c = (my_pos + 1 + N_DEV - g) % N_DEV
                ccw_c = (my_pos + N_DEV - 1 + g) % N_DEV
                cw = pltpu.make_async_remote_copy(
                    src_ref=out_ref.at[top(cw_c), :],
                    dst_ref=out_ref.at[top(cw_c), :],
                    send_sem=cw_ag_s.at[g],
                    recv_sem=cw_ag_r.at[g],
                    device_id=(right,),
                    device_id_type=pl.DeviceIdType.MESH,
                )
                ccw = pltpu.make_async_remote_copy(
                    src_ref=out_ref.at[bot(ccw_c), :],
                    dst_ref=out_ref.at[bot(ccw_c), :],
                    send_sem=ccw_ag_s.at[g],
                    recv_sem=ccw_ag_r.at[g],
                    device_id=(left,),
                    device_id_type=pl.DeviceIdType.MESH,
                )
                cw.start()
                ccw.start()
                cw.wait()
                ccw.wait()

    return pl.pallas_call(
        body,
        grid=(R, T),
        out_shape=jax.ShapeDtypeStruct((m, n), jnp.float32),
        in_specs=[
            pl.BlockSpec((m, k), lambda r, t: (0, 0)),
            pl.BlockSpec((k, ht), lambda r, t: (0, t)),
            pl.BlockSpec((ht, n), lambda r, t: (t, 0)),
        ],
        out_specs=pl.BlockSpec((m, n), lambda r, t: (0, 0)),
        scratch_shapes=[
            pltpu.VMEM((N_DEV - 1, half, n), jnp.float32),
            pltpu.VMEM((N_DEV - 1, half, n), jnp.float32),
            pltpu.SemaphoreType.DMA((N_DEV - 1,)),
            pltpu.SemaphoreType.DMA((N_DEV - 1,)),
            pltpu.SemaphoreType.DMA((N_DEV - 1,)),
            pltpu.SemaphoreType.DMA((N_DEV - 1,)),
            pltpu.SemaphoreType.DMA((N_DEV - 1,)),
            pltpu.SemaphoreType.DMA((N_DEV - 1,)),
            pltpu.SemaphoreType.DMA((N_DEV - 1,)),
            pltpu.SemaphoreType.DMA((N_DEV - 1,)),
        ],
        compiler_params=pltpu.CompilerParams(
            collective_id=0,
            vmem_limit_bytes=100 * 1024 * 1024,
            dimension_semantics=("arbitrary", "arbitrary"),
        ),
    )(x, W1, W2)
